# baseline (speedup 1.0000x reference)
import functools

import jax
import jax.numpy as jnp
from jax import lax
from jax.experimental import pallas as pl
from jax.experimental.pallas import tpu as pltpu
from jax.experimental.pallas import tpu_sc as plsc

TOKENS = 32768
DIM = 4096
NUM_EXPERTS = 64
TOP_K = 8
BLOCK = 1024
NUM_BLOCKS = TOKENS // BLOCK

SC_CORES = 2
SC_SUBCORES = 16
SC_LANES = 16
NW = SC_CORES * SC_SUBCORES
N_IDX = TOKENS * TOP_K
CHUNK = N_IDX // NW


def _router_block_kernel(x_ref, gwt_ref, bias_ref, ts_ref, idx_ref):
    gwt = gwt_ref[...]
    scores = jnp.dot(x_ref[...], gwt, preferred_element_type=jnp.float32)
    scores = jax.nn.sigmoid(scores)
    biased = scores + bias_ref[...]

    col = jax.lax.broadcasted_iota(jnp.int32, (BLOCK, NUM_EXPERTS), 1)
    work = biased
    vals = []
    idxs = []
    for _ in range(TOP_K):
        m = jnp.max(work, axis=1, keepdims=True)
        ix = jnp.min(jnp.where(work == m, col, NUM_EXPERTS), axis=1,
                     keepdims=True)
        onehot = col == ix
        sc = jnp.sum(jnp.where(onehot, scores, 0.0), axis=1, keepdims=True)
        vals.append(sc)
        idxs.append(ix)
        work = jnp.where(onehot, -jnp.inf, work)

    top = jnp.concatenate(vals, axis=1)
    top = top / (jnp.sum(top, axis=1, keepdims=True) + 1e-20)
    ts_ref[...] = top
    idx_ref[...] = jnp.concatenate(idxs, axis=1)


def _sc_hist_kernel(idx_hbm, out_hbm, idx_v, hist_v):
    wid = lax.axis_index("s") * SC_CORES + lax.axis_index("c")
    base = wid * CHUNK
    pltpu.sync_copy(idx_hbm.at[pl.ds(base, CHUNK)], idx_v)
    for j in range(NUM_EXPERTS // SC_LANES):
        hist_v[pl.ds(j * SC_LANES, SC_LANES)] = jnp.zeros(
            (SC_LANES,), jnp.int32)

    def body(i, carry):
        idx = idx_v[pl.ds(pl.multiple_of(i * SC_LANES, SC_LANES), SC_LANES)]
        cnt, last = plsc.scan_count(idx)
        plsc.addupdate_scatter(hist_v, [idx], cnt, mask=last)
        return carry

    lax.fori_loop(0, CHUNK // SC_LANES, body, 0)
    pltpu.sync_copy(hist_v, out_hbm.at[wid])


def _hist_reduce_kernel(parts_ref, out_ref):
    out_ref[...] = jnp.sum(parts_ref[...], axis=0,
                           keepdims=True).astype(jnp.float32)


@jax.jit
def kernel(x, expert_bias, gate_w):
    gwt = gate_w.T
    bias2d = expert_bias.reshape(1, NUM_EXPERTS)

    top_scores, indices = pl.pallas_call(
        _router_block_kernel,
        grid=(NUM_BLOCKS,),
        in_specs=[
            pl.BlockSpec((BLOCK, DIM), lambda i: (i, 0)),
            pl.BlockSpec((DIM, NUM_EXPERTS), lambda i: (0, 0)),
            pl.BlockSpec((1, NUM_EXPERTS), lambda i: (0, 0)),
        ],
        out_specs=[
            pl.BlockSpec((BLOCK, TOP_K), lambda i: (i, 0)),
            pl.BlockSpec((BLOCK, TOP_K), lambda i: (i, 0)),
        ],
        out_shape=[
            jax.ShapeDtypeStruct((TOKENS, TOP_K), jnp.float32),
            jax.ShapeDtypeStruct((TOKENS, TOP_K), jnp.int32),
        ],
        compiler_params=pltpu.CompilerParams(
            dimension_semantics=("parallel",),
        ),
    )(x, gwt, bias2d)

    hist_parts = pl.kernel(
        _sc_hist_kernel,
        out_type=jax.ShapeDtypeStruct((NW, NUM_EXPERTS), jnp.int32),
        mesh=plsc.VectorSubcoreMesh(core_axis_name="c", subcore_axis_name="s"),
        compiler_params=pltpu.CompilerParams(needs_layout_passes=False),
        scratch_types=[
            pltpu.VMEM((CHUNK,), jnp.int32),
            pltpu.VMEM((NUM_EXPERTS,), jnp.int32),
        ],
    )(indices.reshape(N_IDX))

    hist = pl.pallas_call(
        _hist_reduce_kernel,
        out_shape=jax.ShapeDtypeStruct((1, NUM_EXPERTS), jnp.float32),
    )(hist_parts)

    return top_scores, indices, hist.reshape(NUM_EXPERTS)


# speedup vs baseline: 1.1142x; 1.1142x over previous
import jax
import jax.numpy as jnp
from jax.experimental import pallas as pl

TOKENS = 32768
DIM = 4096
NUM_EXPERTS = 64
TOP_K = 8
BLOCK = 1024
NUM_BLOCKS = TOKENS // BLOCK


def _router_block_kernel(x_ref, gwt_ref, bias_ref, ts_ref, idx_ref, hist_ref):
    x_blk = x_ref[...]
    gwt = gwt_ref[...]
    scores = jnp.dot(x_blk, gwt, preferred_element_type=jnp.float32)
    scores = jax.nn.sigmoid(scores)
    biased = scores + bias_ref[...]

    col = jax.lax.broadcasted_iota(jnp.int32, (BLOCK, NUM_EXPERTS), 1)
    work = biased
    sel_mask = jnp.zeros((BLOCK, NUM_EXPERTS), dtype=jnp.float32)
    vals = []
    idxs = []
    for _ in range(TOP_K):
        m = jnp.max(work, axis=1, keepdims=True)
        ix = jnp.min(jnp.where(work == m, col, NUM_EXPERTS), axis=1,
                     keepdims=True)
        onehot = col == ix
        sc = jnp.sum(jnp.where(onehot, scores, 0.0), axis=1, keepdims=True)
        vals.append(sc)
        idxs.append(ix)
        sel_mask = sel_mask + onehot.astype(jnp.float32)
        work = jnp.where(onehot, -jnp.inf, work)

    top = jnp.concatenate(vals, axis=1)
    top = top / (jnp.sum(top, axis=1, keepdims=True) + 1e-20)
    ts_ref[...] = top
    idx_ref[...] = jnp.concatenate(idxs, axis=1)

    part = jnp.sum(sel_mask, axis=0, keepdims=True)
    i = pl.program_id(0)

    @pl.when(i == 0)
    def _():
        hist_ref[...] = part

    @pl.when(i > 0)
    def _():
        hist_ref[...] += part


@jax.jit
def kernel(x, expert_bias, gate_w):
    gwt = gate_w.T
    bias2d = expert_bias.reshape(1, NUM_EXPERTS)

    top_scores, indices, hist = pl.pallas_call(
        _router_block_kernel,
        grid=(NUM_BLOCKS,),
        in_specs=[
            pl.BlockSpec((BLOCK, DIM), lambda i: (i, 0)),
            pl.BlockSpec((DIM, NUM_EXPERTS), lambda i: (0, 0)),
            pl.BlockSpec((1, NUM_EXPERTS), lambda i: (0, 0)),
        ],
        out_specs=[
            pl.BlockSpec((BLOCK, TOP_K), lambda i: (i, 0)),
            pl.BlockSpec((BLOCK, TOP_K), lambda i: (i, 0)),
            pl.BlockSpec((1, NUM_EXPERTS), lambda i: (0, 0)),
        ],
        out_shape=[
            jax.ShapeDtypeStruct((TOKENS, TOP_K), jnp.float32),
            jax.ShapeDtypeStruct((TOKENS, TOP_K), jnp.int32),
            jax.ShapeDtypeStruct((1, NUM_EXPERTS), jnp.float32),
        ],
    )(x, gwt, bias2d)

    return top_scores, indices, hist.reshape(NUM_EXPERTS)
